# baseline (speedup 1.0000x reference)
import functools

import jax
import jax.numpy as jnp
from jax.experimental import pallas as pl
from jax.experimental.pallas import tpu as pltpu


def _split3(a):
    a1 = a.astype(jnp.bfloat16)
    r = a - a1.astype(jnp.float32)
    a2 = r.astype(jnp.bfloat16)
    r2 = r - a2.astype(jnp.float32)
    a3 = r2.astype(jnp.bfloat16)
    return a1, a2, a3


def _dot(a, b):
    return jax.lax.dot_general(
        a, b, (((1,), (0,)), ((), ())),
        preferred_element_type=jnp.float32,
    )


def _tm_kernel(side, std_ref, x_ref, w_ref, out_ref, wa_ref, wb_ref, w2_ref):
    D = x_ref.shape[1]

    @pl.when(pl.program_id(0) == 0)
    def _prep():
        w = w_ref[:]
        w1, w2b, w3 = _split3(w)
        wa_ref[pl.ds(0, D), :] = w1
        wa_ref[pl.ds(D, D), :] = w1
        wa_ref[pl.ds(2 * D, D), :] = w1
        wb_ref[pl.ds(0, D), :] = w2b
        wb_ref[pl.ds(D, D), :] = w3
        wb_ref[pl.ds(2 * D, D), :] = w2b
        w2_ref[:] = jnp.sum(w * w, axis=0, keepdims=True)

    x = x_ref[:]
    s = std_ref[0, 0].astype(jnp.float32)
    inv = 0.5 / (s * s)

    x1, x2b, x3 = _split3(x)
    xa = jnp.concatenate([x1, x2b, x3], axis=1)
    xb = jnp.concatenate([x1, x1, x2b], axis=1)
    xw = _dot(xa, wa_ref[:]) + _dot(xb, wb_ref[:])
    x2 = jnp.sum(x * x, axis=1, keepdims=True)
    n2 = x2 - 2.0 * xw + w2_ref[:]

    mn = jnp.min(n2, axis=1, keepdims=True)
    colid = jax.lax.broadcasted_iota(jnp.int32, n2.shape, 1)
    idx = jnp.min(jnp.where(n2 == mn, colid, n2.shape[1]), axis=1,
                  keepdims=True)

    rowf = (idx // side).astype(jnp.float32)
    colf = (idx % side).astype(jnp.float32)
    gr = (colid // side).astype(jnp.float32)
    gc = (colid % side).astype(jnp.float32)
    dr = gr - rowf
    dc = gc - colf
    phi = jnp.exp(-inv * (dr * dr + dc * dc))
    recip = 1.0 / jnp.sum(phi, axis=1, keepdims=True)
    out_ref[:] = n2 * (phi * recip)


def kernel(x, std, weights):
    B, D = x.shape
    O = weights.shape[1]
    side = int(round(float(O) ** 0.5))
    BB = 1024 if B % 1024 == 0 else B

    std2d = jnp.reshape(jnp.asarray(std), (1, 1))
    body = functools.partial(_tm_kernel, side)
    return pl.pallas_call(
        body,
        grid=(B // BB,),
        in_specs=[
            pl.BlockSpec(memory_space=pltpu.SMEM),
            pl.BlockSpec((BB, D), lambda i: (i, 0)),
            pl.BlockSpec((D, O), lambda i: (0, 0)),
        ],
        out_specs=pl.BlockSpec((BB, O), lambda i: (i, 0)),
        out_shape=jax.ShapeDtypeStruct((B, O), jnp.float32),
        scratch_shapes=[
            pltpu.VMEM((3 * D, O), jnp.bfloat16),
            pltpu.VMEM((3 * D, O), jnp.bfloat16),
            pltpu.VMEM((1, O), jnp.float32),
        ],
    )(std2d, x, weights)


# speedup vs baseline: 1.0625x; 1.0625x over previous
import functools

import jax
import jax.numpy as jnp
from jax.experimental import pallas as pl
from jax.experimental.pallas import tpu as pltpu


def _split3(a):
    a1 = a.astype(jnp.bfloat16)
    r = a - a1.astype(jnp.float32)
    a2 = r.astype(jnp.bfloat16)
    r2 = r - a2.astype(jnp.float32)
    a3 = r2.astype(jnp.bfloat16)
    return a1, a2, a3


def _dot(a, b):
    return jax.lax.dot_general(
        a, b, (((1,), (0,)), ((), ())),
        preferred_element_type=jnp.float32,
    )


def _tm_kernel(side, std_ref, x_ref, w_ref, out_ref,
               wa_ref, wb_ref, w2_ref, gr_ref, gc_ref, g2_ref):
    D = x_ref.shape[1]
    O = out_ref.shape[1]

    @pl.when(pl.program_id(0) == 0)
    def _prep():
        w = w_ref[:]
        w1, w2b, w3 = _split3(w)
        wa_ref[pl.ds(0, D), :] = w1
        wa_ref[pl.ds(D, D), :] = w1
        wa_ref[pl.ds(2 * D, D), :] = w1
        wb_ref[pl.ds(0, D), :] = w2b
        wb_ref[pl.ds(D, D), :] = w3
        wb_ref[pl.ds(2 * D, D), :] = w2b
        w2_ref[:] = jnp.sum(w * w, axis=0, keepdims=True)
        oid = jax.lax.broadcasted_iota(jnp.int32, (1, O), 1)
        grf = (oid // side).astype(jnp.float32)
        gcf = (oid % side).astype(jnp.float32)
        gr_ref[:] = grf
        gc_ref[:] = gcf
        g2_ref[:] = grf * grf + gcf * gcf

    x = x_ref[:]
    BB = x.shape[0]
    s = std_ref[0, 0].astype(jnp.float32)
    inv = 0.5 / (s * s)

    x1, x2b, x3 = _split3(x)
    xa = jnp.concatenate([x1, x2b, x3], axis=1)
    xb = jnp.concatenate([x1, x1, x2b], axis=1)
    xw = _dot(xa, wa_ref[:]) + _dot(xb, wb_ref[:])
    x2 = jnp.sum(x * x, axis=1, keepdims=True)
    n2 = x2 - 2.0 * xw + w2_ref[:]

    mn = jnp.min(n2, axis=1, keepdims=True)
    colid = jax.lax.broadcasted_iota(jnp.int32, n2.shape, 1)
    idx = jnp.min(jnp.where(n2 == mn, colid, O), axis=1,
                  keepdims=True)

    rowf = (idx // side).astype(jnp.float32)
    colf = (idx % side).astype(jnp.float32)
    t = jax.lax.broadcasted_iota(jnp.int32, (BB, side), 1).astype(jnp.float32)
    er = jnp.exp(-inv * (t - rowf) ** 2)
    ec = jnp.exp(-inv * (t - colf) ** 2)
    lrecip = -jnp.log(jnp.sum(er, axis=1, keepdims=True)
                      * jnp.sum(ec, axis=1, keepdims=True))
    u = (2.0 * inv) * rowf
    v = (2.0 * inv) * colf
    k = lrecip - inv * (rowf * rowf + colf * colf)
    arg = (k - inv * g2_ref[:]) + u * gr_ref[:] + v * gc_ref[:]
    out_ref[:] = n2 * jnp.exp(arg)


def kernel(x, std, weights):
    B, D = x.shape
    O = weights.shape[1]
    side = int(round(float(O) ** 0.5))
    BB = 512 if B % 512 == 0 else B

    std2d = jnp.reshape(jnp.asarray(std), (1, 1))
    body = functools.partial(_tm_kernel, side)
    return pl.pallas_call(
        body,
        grid=(B // BB,),
        in_specs=[
            pl.BlockSpec(memory_space=pltpu.SMEM),
            pl.BlockSpec((BB, D), lambda i: (i, 0)),
            pl.BlockSpec((D, O), lambda i: (0, 0)),
        ],
        out_specs=pl.BlockSpec((BB, O), lambda i: (i, 0)),
        out_shape=jax.ShapeDtypeStruct((B, O), jnp.float32),
        scratch_shapes=[
            pltpu.VMEM((3 * D, O), jnp.bfloat16),
            pltpu.VMEM((3 * D, O), jnp.bfloat16),
            pltpu.VMEM((1, O), jnp.float32),
            pltpu.VMEM((1, O), jnp.float32),
            pltpu.VMEM((1, O), jnp.float32),
            pltpu.VMEM((1, O), jnp.float32),
        ],
    )(std2d, x, weights)


# speedup vs baseline: 1.0714x; 1.0084x over previous
import functools

import jax
import jax.numpy as jnp
from jax.experimental import pallas as pl
from jax.experimental.pallas import tpu as pltpu


def _split3(a):
    a1 = a.astype(jnp.bfloat16)
    r = a - a1.astype(jnp.float32)
    a2 = r.astype(jnp.bfloat16)
    r2 = r - a2.astype(jnp.float32)
    a3 = r2.astype(jnp.bfloat16)
    return a1, a2, a3


def _dot(a, b):
    return jax.lax.dot_general(
        a, b, (((1,), (0,)), ((), ())),
        preferred_element_type=jnp.float32,
    )


def _tm_kernel(side, std_ref, x_ref, w_ref, out_ref,
               wa_ref, wb_ref, gr_ref, gc_ref, g2_ref):
    D = x_ref.shape[1]
    O = out_ref.shape[1]

    @pl.when(pl.program_id(0) == 0)
    def _prep():
        w = w_ref[:]
        w1, w2b, w3 = _split3(w)
        n2w1 = (w1.astype(jnp.float32) * -2.0).astype(jnp.bfloat16)
        wa_ref[pl.ds(0, D), :] = n2w1
        wa_ref[pl.ds(D, D), :] = n2w1
        wa_ref[pl.ds(2 * D, D), :] = n2w1
        wsq1, wsq2, wsq3 = _split3(jnp.sum(w * w, axis=0, keepdims=True))
        wa_ref[pl.ds(3 * D, 1), :] = wsq1
        wa_ref[pl.ds(3 * D + 1, 1), :] = wsq2
        wa_ref[pl.ds(3 * D + 2, 1), :] = wsq3
        wb_ref[pl.ds(0, D), :] = (w2b.astype(jnp.float32) * -2.0
                                  ).astype(jnp.bfloat16)
        wb_ref[pl.ds(D, D), :] = (w3.astype(jnp.float32) * -2.0
                                  ).astype(jnp.bfloat16)
        wb_ref[pl.ds(2 * D, D), :] = (w2b.astype(jnp.float32) * -2.0
                                      ).astype(jnp.bfloat16)
        oid = jax.lax.broadcasted_iota(jnp.int32, (1, O), 1)
        grf = (oid // side).astype(jnp.float32)
        gcf = (oid % side).astype(jnp.float32)
        gr_ref[:] = grf
        gc_ref[:] = gcf
        g2_ref[:] = grf * grf + gcf * gcf

    x = x_ref[:]
    BB = x.shape[0]
    s = std_ref[0, 0].astype(jnp.float32)
    inv = 0.5 / (s * s)

    x1, x2b, x3 = _split3(x)
    ones = jnp.ones((BB, 3), dtype=jnp.bfloat16)
    xa = jnp.concatenate([x1, x2b, x3, ones], axis=1)
    xb = jnp.concatenate([x1, x1, x2b], axis=1)
    m2 = _dot(xa, wa_ref[:]) + _dot(xb, wb_ref[:])
    x2 = jnp.sum(x * x, axis=1, keepdims=True)

    idx = jnp.argmin(m2, axis=1, keepdims=True)

    rowf = (idx // side).astype(jnp.float32)
    colf = (idx % side).astype(jnp.float32)
    t = jax.lax.broadcasted_iota(jnp.int32, (BB, side), 1).astype(jnp.float32)
    er = jnp.exp(-inv * (t - rowf) ** 2)
    ec = jnp.exp(-inv * (t - colf) ** 2)
    lrecip = -jnp.log(jnp.sum(er, axis=1, keepdims=True)
                      * jnp.sum(ec, axis=1, keepdims=True))
    u = (2.0 * inv) * rowf
    v = (2.0 * inv) * colf
    k = lrecip - inv * (rowf * rowf + colf * colf)
    arg = (k - inv * g2_ref[:]) + u * gr_ref[:] + v * gc_ref[:]
    out_ref[:] = (m2 + x2) * jnp.exp(arg)


def kernel(x, std, weights):
    B, D = x.shape
    O = weights.shape[1]
    side = int(round(float(O) ** 0.5))
    BB = 512 if B % 512 == 0 else B

    std2d = jnp.reshape(jnp.asarray(std), (1, 1))
    body = functools.partial(_tm_kernel, side)
    return pl.pallas_call(
        body,
        grid=(B // BB,),
        in_specs=[
            pl.BlockSpec(memory_space=pltpu.SMEM),
            pl.BlockSpec((BB, D), lambda i: (i, 0)),
            pl.BlockSpec((D, O), lambda i: (0, 0)),
        ],
        out_specs=pl.BlockSpec((BB, O), lambda i: (i, 0)),
        out_shape=jax.ShapeDtypeStruct((B, O), jnp.float32),
        scratch_shapes=[
            pltpu.VMEM((3 * D + 3, O), jnp.bfloat16),
            pltpu.VMEM((3 * D, O), jnp.bfloat16),
            pltpu.VMEM((1, O), jnp.float32),
            pltpu.VMEM((1, O), jnp.float32),
            pltpu.VMEM((1, O), jnp.float32),
        ],
    )(std2d, x, weights)
